# baseline (device time: 18630 ns/iter reference)
import os

import jax
import jax.numpy as jnp
from jax import lax
from jax.experimental import pallas as pl
from jax.experimental.pallas import tpu as pltpu

N_DEV = 4
N_LAYERS = 3
CH = 4

_ABLATE = os.environ.get("SCB_ABLATE", "")


def kernel(x, Win0, Wout0, Win1, Wout1, Win2, Wout2):
    m, d = x.shape
    _, f = Win0.shape
    dch = d // CH

    def body(x_ref, win0_ref, wout0_ref, win1_ref, wout1_ref, win2_ref,
             wout2_ref, out_ref, win_vmem, wout_vmem, comm_ref,
             load_sems, send_sems, recv_sems):
        my_pos = lax.axis_index("i")

        win_hbm = [win0_ref, win1_ref, win2_ref]
        wout_hbm = [wout0_ref, wout1_ref, wout2_ref]
        loads = []
        for k in range(N_LAYERS):
            ci = pltpu.make_async_copy(
                win_hbm[k], win_vmem.at[k], load_sems.at[2 * k])
            ci.start()
            co = pltpu.make_async_copy(
                wout_hbm[k], wout_vmem.at[k], load_sems.at[2 * k + 1])
            co.start()
            loads.append((ci, co))

        if _ABLATE != "nocomm":
            barrier_sem = pltpu.get_barrier_semaphore()
            for off in range(1, N_DEV):
                peer = (my_pos + off) % N_DEV
                pl.semaphore_signal(
                    barrier_sem, inc=1,
                    device_id=(peer,), device_id_type=pl.DeviceIdType.MESH,
                )
            pl.semaphore_wait(barrier_sem, N_DEV - 1)

        xb = x_ref[...].astype(jnp.bfloat16)
        loads[0][0].wait()
        wi = win_vmem[0].astype(jnp.bfloat16)
        loads[0][1].wait()
        wo = wout_vmem[0].astype(jnp.bfloat16)

        g = jnp.dot(xb, wi, preferred_element_type=jnp.float32)
        sends = []
        for k in range(N_LAYERS):
            h = jnp.maximum(g, 0.0).astype(jnp.bfloat16)

            own = []
            for c in range(CH):
                if _ABLATE == "nocompute":
                    p_c = x_ref[:, c * dch:(c + 1) * dch]
                else:
                    p_c = jnp.dot(h, wo[:, c * dch:(c + 1) * dch],
                                  preferred_element_type=jnp.float32)
                own.append(p_c)
                comm_ref[k, my_pos, c] = p_c.astype(jnp.bfloat16)
                if _ABLATE == "nocomm":
                    continue
                for off in (2, 1, 3):
                    peer = (my_pos + off) % N_DEV
                    rdma = pltpu.make_async_remote_copy(
                        src_ref=comm_ref.at[k, my_pos, c],
                        dst_ref=comm_ref.at[k, my_pos, c],
                        send_sem=send_sems.at[k, off - 1, c],
                        recv_sem=recv_sems.at[k, my_pos, c],
                        device_id=(peer,),
                        device_id_type=pl.DeviceIdType.MESH,
                    )
                    rdma.start()
                    sends.append(rdma)

            if k + 1 < N_LAYERS:
                loads[k + 1][0].wait()
                wi = win_vmem[k + 1].astype(jnp.bfloat16)
                loads[k + 1][1].wait()
                wo = wout_vmem[k + 1].astype(jnp.bfloat16)

            g = None
            for c in range(CH):
                if _ABLATE == "nocomm":
                    x_c = own[c]
                else:
                    for off in range(1, N_DEV):
                        sender = (my_pos + off) % N_DEV
                        recv = pltpu.make_async_remote_copy(
                            src_ref=comm_ref.at[k, sender, c],
                            dst_ref=comm_ref.at[k, sender, c],
                            send_sem=send_sems.at[k, off - 1, c],
                            recv_sem=recv_sems.at[k, sender, c],
                            device_id=(my_pos,),
                            device_id_type=pl.DeviceIdType.MESH,
                        )
                        recv.wait_recv()
                    x_c = jnp.sum(comm_ref[k, :, c].astype(jnp.float32),
                                  axis=0)
                if k + 1 < N_LAYERS:
                    contrib = jnp.dot(
                        x_c.astype(jnp.bfloat16),
                        wi[c * dch:(c + 1) * dch, :],
                        preferred_element_type=jnp.float32,
                    )
                    g = contrib if g is None else g + contrib
                else:
                    out_ref[:, c * dch:(c + 1) * dch] = x_c

        for rdma in sends:
            rdma.wait_send()

    hbm = lambda w: pltpu.with_memory_space_constraint(w, pltpu.MemorySpace.HBM)
    Win0, Wout0 = hbm(Win0), hbm(Wout0)
    Win1, Wout1 = hbm(Win1), hbm(Wout1)
    Win2, Wout2 = hbm(Win2), hbm(Wout2)

    return pl.pallas_call(
        body,
        out_shape=jax.ShapeDtypeStruct((m, d), jnp.float32),
        in_specs=[pl.BlockSpec(memory_space=pltpu.VMEM)]
        + [pl.BlockSpec(memory_space=pltpu.MemorySpace.HBM)] * 6,
        out_specs=pl.BlockSpec(memory_space=pltpu.VMEM),
        scratch_shapes=[
            pltpu.VMEM((N_LAYERS, d, f), jnp.float32),
            pltpu.VMEM((N_LAYERS, f, d), jnp.float32),
            pltpu.VMEM((N_LAYERS, N_DEV, CH, m, dch), jnp.bfloat16),
            pltpu.SemaphoreType.DMA((2 * N_LAYERS,)),
            pltpu.SemaphoreType.DMA((N_LAYERS, N_DEV - 1, CH)),
            pltpu.SemaphoreType.DMA((N_LAYERS, N_DEV, CH)),
        ],
        compiler_params=(
            pltpu.CompilerParams()
            if _ABLATE == "nocomm"
            else pltpu.CompilerParams(collective_id=0)
        ),
    )(x, Win0, Wout0, Win1, Wout1, Win2, Wout2)


# device time: 17865 ns/iter; 1.0428x vs baseline; 1.0428x over previous
import os

import jax
import jax.numpy as jnp
from jax import lax
from jax.experimental import pallas as pl
from jax.experimental.pallas import tpu as pltpu

N_DEV = 4
N_LAYERS = 3
CH = 2

_ABLATE = os.environ.get("SCB_ABLATE", "")


def kernel(x, Win0, Wout0, Win1, Wout1, Win2, Wout2):
    m, d = x.shape
    _, f = Win0.shape
    dch = d // CH

    def body(x_ref, win0_ref, wout0_ref, win1_ref, wout1_ref, win2_ref,
             wout2_ref, out_ref, win_vmem, wout_vmem, comm_ref, out_vmem,
             load_sems, w0_sems, out_sems, send_sems, recv_sems):
        my_pos = lax.axis_index("i")

        win_hbm = [win0_ref, win1_ref, win2_ref]
        wout_hbm = [wout0_ref, wout1_ref, wout2_ref]
        fh = f // 2
        w0_loads = []
        for c in range(2):
            cp = pltpu.make_async_copy(
                win_hbm[0].at[:, c * fh:(c + 1) * fh],
                win_vmem.at[0, :, c * fh:(c + 1) * fh],
                w0_sems.at[c])
            cp.start()
            w0_loads.append(cp)
        loads = []
        for k in range(N_LAYERS):
            ci = None
            if k > 0:
                ci = pltpu.make_async_copy(
                    win_hbm[k], win_vmem.at[k], load_sems.at[2 * k])
                ci.start()
            co = pltpu.make_async_copy(
                wout_hbm[k], wout_vmem.at[k], load_sems.at[2 * k + 1])
            co.start()
            loads.append((ci, co))

        if _ABLATE != "nocomm":
            barrier_sem = pltpu.get_barrier_semaphore()
            for off in range(1, N_DEV):
                peer = (my_pos + off) % N_DEV
                pl.semaphore_signal(
                    barrier_sem, inc=1,
                    device_id=(peer,), device_id_type=pl.DeviceIdType.MESH,
                )
            pl.semaphore_wait(barrier_sem, N_DEV - 1)

        xb = x_ref[...].astype(jnp.bfloat16)
        g_halves = []
        for c in range(2):
            w0_loads[c].wait()
            wi_h = win_vmem[0, :, c * fh:(c + 1) * fh].astype(jnp.bfloat16)
            g_halves.append(
                jnp.dot(xb, wi_h, preferred_element_type=jnp.float32))
        g = jnp.concatenate(g_halves, axis=1)
        loads[0][1].wait()
        wo = wout_vmem[0].astype(jnp.bfloat16)
        sends = []
        out_cps = []
        for k in range(N_LAYERS):
            h = jnp.maximum(g, 0.0).astype(jnp.bfloat16)

            own = []
            for c in range(CH):
                if _ABLATE == "nocompute":
                    p_c = x_ref[:, c * dch:(c + 1) * dch]
                else:
                    p_c = jnp.dot(h, wo[:, c * dch:(c + 1) * dch],
                                  preferred_element_type=jnp.float32)
                own.append(p_c)
                comm_ref[k, my_pos, c] = p_c.astype(jnp.bfloat16)
                if _ABLATE == "nocomm":
                    continue
                for off in (2, 1, 3):
                    peer = (my_pos + off) % N_DEV
                    rdma = pltpu.make_async_remote_copy(
                        src_ref=comm_ref.at[k, my_pos, c],
                        dst_ref=comm_ref.at[k, my_pos, c],
                        send_sem=send_sems.at[k, off - 1, c],
                        recv_sem=recv_sems.at[k, my_pos, c],
                        device_id=(peer,),
                        device_id_type=pl.DeviceIdType.MESH,
                    )
                    rdma.start()
                    sends.append(rdma)

            if k + 1 < N_LAYERS:
                loads[k + 1][0].wait()
                wi = win_vmem[k + 1].astype(jnp.bfloat16)
                loads[k + 1][1].wait()
                wo = wout_vmem[k + 1].astype(jnp.bfloat16)

            g = None
            for c in range(CH):
                if _ABLATE == "nocomm":
                    x_c = own[c]
                else:
                    for off in range(1, N_DEV):
                        sender = (my_pos + off) % N_DEV
                        recv = pltpu.make_async_remote_copy(
                            src_ref=comm_ref.at[k, sender, c],
                            dst_ref=comm_ref.at[k, sender, c],
                            send_sem=send_sems.at[k, off - 1, c],
                            recv_sem=recv_sems.at[k, sender, c],
                            device_id=(my_pos,),
                            device_id_type=pl.DeviceIdType.MESH,
                        )
                        recv.wait_recv()
                    x_c = jnp.sum(comm_ref[k, :, c].astype(jnp.float32),
                                  axis=0)
                if k + 1 < N_LAYERS:
                    contrib = jnp.dot(
                        x_c.astype(jnp.bfloat16),
                        wi[c * dch:(c + 1) * dch, :],
                        preferred_element_type=jnp.float32,
                    )
                    g = contrib if g is None else g + contrib
                else:
                    out_vmem[:, c * dch:(c + 1) * dch] = x_c
                    ocp = pltpu.make_async_copy(
                        out_vmem.at[:, c * dch:(c + 1) * dch],
                        out_ref.at[:, c * dch:(c + 1) * dch],
                        out_sems.at[c])
                    ocp.start()
                    out_cps.append(ocp)

        for ocp in out_cps:
            ocp.wait()
        for rdma in sends:
            rdma.wait_send()

    hbm = lambda w: pltpu.with_memory_space_constraint(w, pltpu.MemorySpace.HBM)
    Win0, Wout0 = hbm(Win0), hbm(Wout0)
    Win1, Wout1 = hbm(Win1), hbm(Wout1)
    Win2, Wout2 = hbm(Win2), hbm(Wout2)

    return pl.pallas_call(
        body,
        out_shape=jax.ShapeDtypeStruct((m, d), jnp.float32),
        in_specs=[pl.BlockSpec(memory_space=pltpu.VMEM)]
        + [pl.BlockSpec(memory_space=pltpu.MemorySpace.HBM)] * 6,
        out_specs=pl.BlockSpec(memory_space=pltpu.MemorySpace.HBM),
        scratch_shapes=[
            pltpu.VMEM((N_LAYERS, d, f), jnp.float32),
            pltpu.VMEM((N_LAYERS, f, d), jnp.float32),
            pltpu.VMEM((N_LAYERS, N_DEV, CH, m, dch), jnp.bfloat16),
            pltpu.VMEM((m, d), jnp.float32),
            pltpu.SemaphoreType.DMA((2 * N_LAYERS,)),
            pltpu.SemaphoreType.DMA((2,)),
            pltpu.SemaphoreType.DMA((CH,)),
            pltpu.SemaphoreType.DMA((N_LAYERS, N_DEV - 1, CH)),
            pltpu.SemaphoreType.DMA((N_LAYERS, N_DEV, CH)),
        ],
        compiler_params=(
            pltpu.CompilerParams()
            if _ABLATE == "nocomm"
            else pltpu.CompilerParams(collective_id=0)
        ),
    )(x, Win0, Wout0, Win1, Wout1, Win2, Wout2)


# device time: 16959 ns/iter; 1.0985x vs baseline; 1.0534x over previous
import os

import jax
import jax.numpy as jnp
from jax import lax
from jax.experimental import pallas as pl
from jax.experimental.pallas import tpu as pltpu

N_DEV = 4
N_LAYERS = 3
CH = 2

_ABLATE = os.environ.get("SCB_ABLATE", "")


def kernel(x, Win0, Wout0, Win1, Wout1, Win2, Wout2):
    m, d = x.shape
    _, f = Win0.shape
    dch = d // CH

    def body(x_ref, win0_ref, wout0_ref, win1_ref, wout1_ref, win2_ref,
             wout2_ref, out_ref, x_vmem, win_vmem, wout_vmem, comm_ref,
             out_vmem, load_sems, w0_sems, out_sems, send_sems, recv_sems):
        my_pos = lax.axis_index("i")

        x_cp = pltpu.make_async_copy(x_ref, x_vmem, load_sems.at[0])
        x_cp.start()

        win_hbm = [win0_ref, win1_ref, win2_ref]
        wout_hbm = [wout0_ref, wout1_ref, wout2_ref]
        fh = f // 2
        w0_loads = []
        for c in range(2):
            cp = pltpu.make_async_copy(
                win_hbm[0].at[:, c * fh:(c + 1) * fh],
                win_vmem.at[0, :, c * fh:(c + 1) * fh],
                w0_sems.at[c])
            cp.start()
            w0_loads.append(cp)
        loads = []
        for k in range(N_LAYERS):
            ci = None
            if k > 0:
                ci = pltpu.make_async_copy(
                    win_hbm[k], win_vmem.at[k], load_sems.at[2 * k])
                ci.start()
            co = pltpu.make_async_copy(
                wout_hbm[k], wout_vmem.at[k], load_sems.at[2 * k + 1])
            co.start()
            loads.append((ci, co))

        if _ABLATE != "nocomm":
            barrier_sem = pltpu.get_barrier_semaphore()
            for off in range(1, N_DEV):
                peer = (my_pos + off) % N_DEV
                pl.semaphore_signal(
                    barrier_sem, inc=1,
                    device_id=(peer,), device_id_type=pl.DeviceIdType.MESH,
                )
            pl.semaphore_wait(barrier_sem, N_DEV - 1)

        x_cp.wait()
        xb = x_vmem[...].astype(jnp.bfloat16)
        g_halves = []
        for c in range(2):
            w0_loads[c].wait()
            wi_h = win_vmem[0, :, c * fh:(c + 1) * fh].astype(jnp.bfloat16)
            g_halves.append(
                jnp.dot(xb, wi_h, preferred_element_type=jnp.float32))
        g = jnp.concatenate(g_halves, axis=1)
        loads[0][1].wait()
        wo = wout_vmem[0].astype(jnp.bfloat16)
        sends = []
        out_cps = []
        for k in range(N_LAYERS):
            h = jnp.maximum(g, 0.0).astype(jnp.bfloat16)

            own = []
            for c in range(CH):
                if _ABLATE == "nocompute":
                    p_c = x_vmem[:, c * dch:(c + 1) * dch]
                else:
                    p_c = jnp.dot(h, wo[:, c * dch:(c + 1) * dch],
                                  preferred_element_type=jnp.float32)
                own.append(p_c)
                comm_ref[k, my_pos, c] = p_c.astype(jnp.bfloat16)
                if _ABLATE == "nocomm":
                    continue
                for off in (2, 1, 3):
                    peer = (my_pos + off) % N_DEV
                    rdma = pltpu.make_async_remote_copy(
                        src_ref=comm_ref.at[k, my_pos, c],
                        dst_ref=comm_ref.at[k, my_pos, c],
                        send_sem=send_sems.at[k, off - 1, c],
                        recv_sem=recv_sems.at[k, my_pos, c],
                        device_id=(peer,),
                        device_id_type=pl.DeviceIdType.MESH,
                    )
                    rdma.start()
                    sends.append(rdma)

            if k + 1 < N_LAYERS:
                loads[k + 1][0].wait()
                wi = win_vmem[k + 1].astype(jnp.bfloat16)
                loads[k + 1][1].wait()
                wo = wout_vmem[k + 1].astype(jnp.bfloat16)

            g = None
            for c in range(CH):
                if _ABLATE == "nocomm":
                    x_c = own[c]
                else:
                    for off in range(1, N_DEV):
                        sender = (my_pos + off) % N_DEV
                        recv = pltpu.make_async_remote_copy(
                            src_ref=comm_ref.at[k, sender, c],
                            dst_ref=comm_ref.at[k, sender, c],
                            send_sem=send_sems.at[k, off - 1, c],
                            recv_sem=recv_sems.at[k, sender, c],
                            device_id=(my_pos,),
                            device_id_type=pl.DeviceIdType.MESH,
                        )
                        recv.wait_recv()
                    x_c = jnp.sum(comm_ref[k, :, c].astype(jnp.float32),
                                  axis=0)
                if k + 1 < N_LAYERS:
                    contrib = jnp.dot(
                        x_c.astype(jnp.bfloat16),
                        wi[c * dch:(c + 1) * dch, :],
                        preferred_element_type=jnp.float32,
                    )
                    g = contrib if g is None else g + contrib
                else:
                    out_vmem[:, c * dch:(c + 1) * dch] = x_c
                    ocp = pltpu.make_async_copy(
                        out_vmem.at[:, c * dch:(c + 1) * dch],
                        out_ref.at[:, c * dch:(c + 1) * dch],
                        out_sems.at[c])
                    ocp.start()
                    out_cps.append(ocp)

        for ocp in out_cps:
            ocp.wait()
        for rdma in sends:
            rdma.wait_send()

    hbm = lambda w: pltpu.with_memory_space_constraint(w, pltpu.MemorySpace.HBM)
    x = hbm(x)
    Win0, Wout0 = hbm(Win0), hbm(Wout0)
    Win1, Wout1 = hbm(Win1), hbm(Wout1)
    Win2, Wout2 = hbm(Win2), hbm(Wout2)

    return pl.pallas_call(
        body,
        out_shape=jax.ShapeDtypeStruct((m, d), jnp.float32),
        in_specs=[pl.BlockSpec(memory_space=pltpu.MemorySpace.HBM)] * 7,
        out_specs=pl.BlockSpec(memory_space=pltpu.MemorySpace.HBM),
        scratch_shapes=[
            pltpu.VMEM((m, d), jnp.float32),
            pltpu.VMEM((N_LAYERS, d, f), jnp.float32),
            pltpu.VMEM((N_LAYERS, f, d), jnp.float32),
            pltpu.VMEM((N_LAYERS, N_DEV, CH, m, dch), jnp.bfloat16),
            pltpu.VMEM((m, d), jnp.float32),
            pltpu.SemaphoreType.DMA((2 * N_LAYERS,)),
            pltpu.SemaphoreType.DMA((2,)),
            pltpu.SemaphoreType.DMA((CH,)),
            pltpu.SemaphoreType.DMA((N_LAYERS, N_DEV - 1, CH)),
            pltpu.SemaphoreType.DMA((N_LAYERS, N_DEV, CH)),
        ],
        compiler_params=(
            pltpu.CompilerParams()
            if _ABLATE == "nocomm"
            else pltpu.CompilerParams(collective_id=0)
        ),
    )(x, Win0, Wout0, Win1, Wout1, Win2, Wout2)


# device time: 16924 ns/iter; 1.1008x vs baseline; 1.0021x over previous
import os

import jax
import jax.numpy as jnp
from jax import lax
from jax.experimental import pallas as pl
from jax.experimental.pallas import tpu as pltpu

N_DEV = 4
N_LAYERS = 3
CH = 2

_ABLATE = os.environ.get("SCB_ABLATE", "")


def kernel(x, Win0, Wout0, Win1, Wout1, Win2, Wout2):
    m, d = x.shape
    _, f = Win0.shape
    dch = d // CH

    def body(x_ref, win0_ref, wout0_ref, win1_ref, wout1_ref, win2_ref,
             wout2_ref, out_ref, x_vmem, win_vmem, wout_vmem, comm_ref,
             out_vmem, load_sems, w0_sems, out_sems, send_sems, recv_sems):
        my_pos = lax.axis_index("i")

        x_cp = pltpu.make_async_copy(x_ref, x_vmem, load_sems.at[0])
        x_cp.start()

        win_hbm = [win0_ref, win1_ref, win2_ref]
        wout_hbm = [wout0_ref, wout1_ref, wout2_ref]
        fh = f // 2
        w0_loads = []
        for c in range(2):
            cp = pltpu.make_async_copy(
                win_hbm[0].at[:, c * fh:(c + 1) * fh],
                win_vmem.at[0, :, c * fh:(c + 1) * fh],
                w0_sems.at[c])
            cp.start()
            w0_loads.append(cp)
        loads = []
        for k in range(N_LAYERS):
            ci = None
            if k > 0:
                ci = pltpu.make_async_copy(
                    win_hbm[k], win_vmem.at[k], load_sems.at[2 * k])
                ci.start()
            co = pltpu.make_async_copy(
                wout_hbm[k], wout_vmem.at[k], load_sems.at[2 * k + 1])
            co.start()
            loads.append((ci, co))

        if _ABLATE != "nocomm":
            barrier_sem = pltpu.get_barrier_semaphore()
            for off in range(1, N_DEV):
                peer = (my_pos + off) % N_DEV
                pl.semaphore_signal(
                    barrier_sem, inc=1,
                    device_id=(peer,), device_id_type=pl.DeviceIdType.MESH,
                )
            pl.semaphore_wait(barrier_sem, N_DEV - 1)

        x_cp.wait()
        xb = x_vmem[...].astype(jnp.bfloat16)
        g_halves = []
        for c in range(2):
            w0_loads[c].wait()
            wi_h = win_vmem[0, :, c * fh:(c + 1) * fh].astype(jnp.bfloat16)
            g_halves.append(
                jnp.dot(xb, wi_h, preferred_element_type=jnp.float32))
        g = jnp.concatenate(g_halves, axis=1)
        loads[0][1].wait()
        wo = wout_vmem[0].astype(jnp.bfloat16)
        sends = []
        out_cps = []
        for k in range(N_LAYERS):
            h = jnp.maximum(g, 0.0).astype(jnp.bfloat16)

            own = []
            for c in range(CH):
                if _ABLATE == "nocompute":
                    p_c = x_vmem[:, c * dch:(c + 1) * dch]
                else:
                    p_c = jnp.dot(h, wo[:, c * dch:(c + 1) * dch],
                                  preferred_element_type=jnp.float32)
                own.append(p_c)
                comm_ref[k, my_pos, c] = p_c.astype(jnp.bfloat16)
                if _ABLATE == "nocomm":
                    continue
                for off in (2, 1, 3):
                    peer = (my_pos + off) % N_DEV
                    rdma = pltpu.make_async_remote_copy(
                        src_ref=comm_ref.at[k, my_pos, c],
                        dst_ref=comm_ref.at[k, my_pos, c],
                        send_sem=send_sems.at[k, off - 1, c],
                        recv_sem=recv_sems.at[k, my_pos, c],
                        device_id=(peer,),
                        device_id_type=pl.DeviceIdType.MESH,
                    )
                    rdma.start()
                    sends.append(rdma)

            if k + 1 < N_LAYERS:
                loads[k + 1][0].wait()
                wi = win_vmem[k + 1].astype(jnp.bfloat16)
                loads[k + 1][1].wait()
                wo = wout_vmem[k + 1].astype(jnp.bfloat16)

            g = None
            for c in range(CH):
                if _ABLATE == "nocomm":
                    x_c = own[c]
                else:
                    for off in range(1, N_DEV):
                        sender = (my_pos + off) % N_DEV
                        recv = pltpu.make_async_remote_copy(
                            src_ref=comm_ref.at[k, sender, c],
                            dst_ref=comm_ref.at[k, sender, c],
                            send_sem=send_sems.at[k, off - 1, c],
                            recv_sem=recv_sems.at[k, sender, c],
                            device_id=(my_pos,),
                            device_id_type=pl.DeviceIdType.MESH,
                        )
                        recv.wait_recv()
                    x_c = jnp.sum(comm_ref[k, :, c].astype(jnp.float32),
                                  axis=0)
                if k + 1 < N_LAYERS:
                    contrib = jnp.dot(
                        x_c.astype(jnp.bfloat16),
                        wi[c * dch:(c + 1) * dch, :],
                        preferred_element_type=jnp.float32,
                    )
                    g = contrib if g is None else g + contrib
                else:
                    out_vmem[:, c * dch:(c + 1) * dch] = x_c.astype(jnp.bfloat16)
                    ocp = pltpu.make_async_copy(
                        out_vmem.at[:, c * dch:(c + 1) * dch],
                        out_ref.at[:, c * dch:(c + 1) * dch],
                        out_sems.at[c])
                    ocp.start()
                    out_cps.append(ocp)

        for ocp in out_cps:
            ocp.wait()
        for rdma in sends:
            rdma.wait_send()

    hbm = lambda w: pltpu.with_memory_space_constraint(w, pltpu.MemorySpace.HBM)
    x = hbm(x)
    Win0, Wout0 = hbm(Win0), hbm(Wout0)
    Win1, Wout1 = hbm(Win1), hbm(Wout1)
    Win2, Wout2 = hbm(Win2), hbm(Wout2)

    return pl.pallas_call(
        body,
        out_shape=jax.ShapeDtypeStruct((m, d), jnp.bfloat16),
        in_specs=[pl.BlockSpec(memory_space=pltpu.MemorySpace.HBM)] * 7,
        out_specs=pl.BlockSpec(memory_space=pltpu.MemorySpace.HBM),
        scratch_shapes=[
            pltpu.VMEM((m, d), jnp.float32),
            pltpu.VMEM((N_LAYERS, d, f), jnp.float32),
            pltpu.VMEM((N_LAYERS, f, d), jnp.float32),
            pltpu.VMEM((N_LAYERS, N_DEV, CH, m, dch), jnp.bfloat16),
            pltpu.VMEM((m, d), jnp.bfloat16),
            pltpu.SemaphoreType.DMA((2 * N_LAYERS,)),
            pltpu.SemaphoreType.DMA((2,)),
            pltpu.SemaphoreType.DMA((CH,)),
            pltpu.SemaphoreType.DMA((N_LAYERS, N_DEV - 1, CH)),
            pltpu.SemaphoreType.DMA((N_LAYERS, N_DEV, CH)),
        ],
        compiler_params=(
            pltpu.CompilerParams()
            if _ABLATE == "nocomm"
            else pltpu.CompilerParams(collective_id=0)
        ),
    )(x, Win0, Wout0, Win1, Wout1, Win2, Wout2)


# device time: 16913 ns/iter; 1.1015x vs baseline; 1.0007x over previous
import os

import jax
import jax.numpy as jnp
from jax import lax
from jax.experimental import pallas as pl
from jax.experimental.pallas import tpu as pltpu

N_DEV = 4
N_LAYERS = 3
CH = 2

_ABLATE = os.environ.get("SCB_ABLATE", "")


def kernel(x, Win0, Wout0, Win1, Wout1, Win2, Wout2):
    m, d = x.shape
    _, f = Win0.shape
    dch = d // CH

    def body(x_ref, win0_ref, wout0_ref, win1_ref, wout1_ref, win2_ref,
             wout2_ref, out_ref, x_vmem, win_vmem, wout_vmem, comm_ref,
             out_vmem, load_sems, w0_sems, out_sems, send_sems, recv_sems):
        my_pos = lax.axis_index("i")

        x_cp = pltpu.make_async_copy(x_ref, x_vmem, load_sems.at[0])
        x_cp.start()

        win_hbm = [win0_ref, win1_ref, win2_ref]
        wout_hbm = [wout0_ref, wout1_ref, wout2_ref]
        fh = f // 2
        w0_loads = []
        for c in range(2):
            cp = pltpu.make_async_copy(
                win_hbm[0].at[:, c * fh:(c + 1) * fh],
                win_vmem.at[0, :, c * fh:(c + 1) * fh],
                w0_sems.at[c])
            cp.start()
            w0_loads.append(cp)
        loads = []
        for k in range(N_LAYERS):
            ci = None
            if k > 0:
                ci = pltpu.make_async_copy(
                    win_hbm[k], win_vmem.at[k], load_sems.at[2 * k])
                ci.start()
            co = pltpu.make_async_copy(
                wout_hbm[k], wout_vmem.at[k], load_sems.at[2 * k + 1])
            co.start()
            loads.append((ci, co))

        if _ABLATE != "nocomm":
            barrier_sem = pltpu.get_barrier_semaphore()
            for off in range(1, N_DEV):
                peer = (my_pos + off) % N_DEV
                pl.semaphore_signal(
                    barrier_sem, inc=1,
                    device_id=(peer,), device_id_type=pl.DeviceIdType.MESH,
                )
            pl.semaphore_wait(barrier_sem, N_DEV - 1)

        x_cp.wait()
        xb = x_vmem[...].astype(jnp.bfloat16)
        g_halves = []
        for c in range(2):
            w0_loads[c].wait()
            wi_h = win_vmem[0, :, c * fh:(c + 1) * fh].astype(jnp.bfloat16)
            g_halves.append(
                jnp.dot(xb, wi_h, preferred_element_type=jnp.float32))
        g = jnp.concatenate(g_halves, axis=1)
        loads[0][1].wait()
        wo = wout_vmem[0].astype(jnp.bfloat16)
        sends = []
        out_cps = []
        for k in range(N_LAYERS):
            h = jnp.maximum(g, 0.0).astype(jnp.bfloat16)

            own = []
            for c in range(CH):
                if _ABLATE == "nocompute":
                    p_c = x_vmem[:, c * dch:(c + 1) * dch]
                else:
                    p_c = jnp.dot(h, wo[:, c * dch:(c + 1) * dch],
                                  preferred_element_type=jnp.float32)
                own.append(p_c)
                comm_ref[k, my_pos, c] = p_c.astype(jnp.bfloat16)
                if _ABLATE == "nocomm":
                    continue
                for off in (2, 1, 3):
                    peer = (my_pos + off) % N_DEV
                    rdma = pltpu.make_async_remote_copy(
                        src_ref=comm_ref.at[k, my_pos, c],
                        dst_ref=comm_ref.at[k, my_pos, c],
                        send_sem=send_sems.at[k, off - 1, c],
                        recv_sem=recv_sems.at[k, my_pos, c],
                        device_id=(peer,),
                        device_id_type=pl.DeviceIdType.MESH,
                    )
                    rdma.start()
                    sends.append(rdma)

            if k + 1 < N_LAYERS:
                loads[k + 1][0].wait()
                wi = win_vmem[k + 1].astype(jnp.bfloat16)
                loads[k + 1][1].wait()
                wo = wout_vmem[k + 1].astype(jnp.bfloat16)

            g = None
            for c in range(CH):
                if _ABLATE == "nocomm":
                    x_c = own[c]
                else:
                    x_c = own[c]
                    for off in (1, 3, 2):
                        sender = (my_pos + off) % N_DEV
                        recv = pltpu.make_async_remote_copy(
                            src_ref=comm_ref.at[k, sender, c],
                            dst_ref=comm_ref.at[k, sender, c],
                            send_sem=send_sems.at[k, off - 1, c],
                            recv_sem=recv_sems.at[k, sender, c],
                            device_id=(my_pos,),
                            device_id_type=pl.DeviceIdType.MESH,
                        )
                        recv.wait_recv()
                        x_c = x_c + comm_ref[k, sender, c].astype(jnp.float32)
                if k + 1 < N_LAYERS:
                    contrib = jnp.dot(
                        x_c.astype(jnp.bfloat16),
                        wi[c * dch:(c + 1) * dch, :],
                        preferred_element_type=jnp.float32,
                    )
                    g = contrib if g is None else g + contrib
                else:
                    out_vmem[:, c * dch:(c + 1) * dch] = x_c.astype(jnp.bfloat16)
                    ocp = pltpu.make_async_copy(
                        out_vmem.at[:, c * dch:(c + 1) * dch],
                        out_ref.at[:, c * dch:(c + 1) * dch],
                        out_sems.at[c])
                    ocp.start()
                    out_cps.append(ocp)

        for ocp in out_cps:
            ocp.wait()
        for rdma in sends:
            rdma.wait_send()

    hbm = lambda w: pltpu.with_memory_space_constraint(w, pltpu.MemorySpace.HBM)
    x = hbm(x)
    Win0, Wout0 = hbm(Win0), hbm(Wout0)
    Win1, Wout1 = hbm(Win1), hbm(Wout1)
    Win2, Wout2 = hbm(Win2), hbm(Wout2)

    return pl.pallas_call(
        body,
        out_shape=jax.ShapeDtypeStruct((m, d), jnp.bfloat16),
        in_specs=[pl.BlockSpec(memory_space=pltpu.MemorySpace.HBM)] * 7,
        out_specs=pl.BlockSpec(memory_space=pltpu.MemorySpace.HBM),
        scratch_shapes=[
            pltpu.VMEM((m, d), jnp.float32),
            pltpu.VMEM((N_LAYERS, d, f), jnp.float32),
            pltpu.VMEM((N_LAYERS, f, d), jnp.float32),
            pltpu.VMEM((N_LAYERS, N_DEV, CH, m, dch), jnp.bfloat16),
            pltpu.VMEM((m, d), jnp.bfloat16),
            pltpu.SemaphoreType.DMA((2 * N_LAYERS,)),
            pltpu.SemaphoreType.DMA((2,)),
            pltpu.SemaphoreType.DMA((CH,)),
            pltpu.SemaphoreType.DMA((N_LAYERS, N_DEV - 1, CH)),
            pltpu.SemaphoreType.DMA((N_LAYERS, N_DEV, CH)),
        ],
        compiler_params=(
            pltpu.CompilerParams()
            if _ABLATE == "nocomm"
            else pltpu.CompilerParams(collective_id=0)
        ),
    )(x, Win0, Wout0, Win1, Wout1, Win2, Wout2)


# device time: 16244 ns/iter; 1.1469x vs baseline; 1.0412x over previous
import os

import jax
import jax.numpy as jnp
from jax import lax
from jax.experimental import pallas as pl
from jax.experimental.pallas import tpu as pltpu

N_DEV = 4
N_LAYERS = 3
CH = 1

_ABLATE = os.environ.get("SCB_ABLATE", "")


def kernel(x, Win0, Wout0, Win1, Wout1, Win2, Wout2):
    m, d = x.shape
    _, f = Win0.shape
    dch = d // CH

    def body(x_ref, win0_ref, wout0_ref, win1_ref, wout1_ref, win2_ref,
             wout2_ref, out_ref, x_vmem, win_vmem, wout_vmem, comm_ref,
             out_vmem, load_sems, w0_sems, out_sems, send_sems, recv_sems):
        my_pos = lax.axis_index("i")

        x_cp = pltpu.make_async_copy(x_ref, x_vmem, load_sems.at[0])
        x_cp.start()

        win_hbm = [win0_ref, win1_ref, win2_ref]
        wout_hbm = [wout0_ref, wout1_ref, wout2_ref]
        fh = f // 2
        w0_loads = []
        for c in range(2):
            cp = pltpu.make_async_copy(
                win_hbm[0].at[:, c * fh:(c + 1) * fh],
                win_vmem.at[0, :, c * fh:(c + 1) * fh],
                w0_sems.at[c])
            cp.start()
            w0_loads.append(cp)
        loads = []
        for k in range(N_LAYERS):
            ci = None
            if k > 0:
                ci = pltpu.make_async_copy(
                    win_hbm[k], win_vmem.at[k], load_sems.at[2 * k])
                ci.start()
            co = pltpu.make_async_copy(
                wout_hbm[k], wout_vmem.at[k], load_sems.at[2 * k + 1])
            co.start()
            loads.append((ci, co))

        if _ABLATE != "nocomm":
            barrier_sem = pltpu.get_barrier_semaphore()
            for off in range(1, N_DEV):
                peer = (my_pos + off) % N_DEV
                pl.semaphore_signal(
                    barrier_sem, inc=1,
                    device_id=(peer,), device_id_type=pl.DeviceIdType.MESH,
                )
            pl.semaphore_wait(barrier_sem, N_DEV - 1)

        x_cp.wait()
        xb = x_vmem[...].astype(jnp.bfloat16)
        g_halves = []
        for c in range(2):
            w0_loads[c].wait()
            wi_h = win_vmem[0, :, c * fh:(c + 1) * fh].astype(jnp.bfloat16)
            g_halves.append(
                jnp.dot(xb, wi_h, preferred_element_type=jnp.float32))
        g = jnp.concatenate(g_halves, axis=1)
        loads[0][1].wait()
        wo = wout_vmem[0].astype(jnp.bfloat16)
        sends = []
        out_cps = []
        for k in range(N_LAYERS):
            h = jnp.maximum(g, 0.0).astype(jnp.bfloat16)

            own = []
            for c in range(CH):
                if _ABLATE == "nocompute":
                    p_c = x_vmem[:, c * dch:(c + 1) * dch]
                else:
                    p_c = jnp.dot(h, wo[:, c * dch:(c + 1) * dch],
                                  preferred_element_type=jnp.float32)
                own.append(p_c)
                comm_ref[k, my_pos, c] = p_c.astype(jnp.bfloat16)
                if _ABLATE == "nocomm":
                    continue
                for off in (2, 1, 3):
                    peer = (my_pos + off) % N_DEV
                    rdma = pltpu.make_async_remote_copy(
                        src_ref=comm_ref.at[k, my_pos, c],
                        dst_ref=comm_ref.at[k, my_pos, c],
                        send_sem=send_sems.at[k, off - 1, c],
                        recv_sem=recv_sems.at[k, my_pos, c],
                        device_id=(peer,),
                        device_id_type=pl.DeviceIdType.MESH,
                    )
                    rdma.start()
                    sends.append(rdma)

            if k + 1 < N_LAYERS:
                loads[k + 1][0].wait()
                wi = win_vmem[k + 1].astype(jnp.bfloat16)
                loads[k + 1][1].wait()
                wo = wout_vmem[k + 1].astype(jnp.bfloat16)

            g = None
            for c in range(CH):
                if _ABLATE == "nocomm":
                    x_c = own[c]
                else:
                    x_c = own[c]
                    for off in (1, 3, 2):
                        sender = (my_pos + off) % N_DEV
                        recv = pltpu.make_async_remote_copy(
                            src_ref=comm_ref.at[k, sender, c],
                            dst_ref=comm_ref.at[k, sender, c],
                            send_sem=send_sems.at[k, off - 1, c],
                            recv_sem=recv_sems.at[k, sender, c],
                            device_id=(my_pos,),
                            device_id_type=pl.DeviceIdType.MESH,
                        )
                        recv.wait_recv()
                        x_c = x_c + comm_ref[k, sender, c].astype(jnp.float32)
                if k + 1 < N_LAYERS:
                    contrib = jnp.dot(
                        x_c.astype(jnp.bfloat16),
                        wi[c * dch:(c + 1) * dch, :],
                        preferred_element_type=jnp.float32,
                    )
                    g = contrib if g is None else g + contrib
                else:
                    out_vmem[:, c * dch:(c + 1) * dch] = x_c.astype(jnp.bfloat16)
                    ocp = pltpu.make_async_copy(
                        out_vmem.at[:, c * dch:(c + 1) * dch],
                        out_ref.at[:, c * dch:(c + 1) * dch],
                        out_sems.at[c])
                    ocp.start()
                    out_cps.append(ocp)

        for ocp in out_cps:
            ocp.wait()
        for rdma in sends:
            rdma.wait_send()

    hbm = lambda w: pltpu.with_memory_space_constraint(w, pltpu.MemorySpace.HBM)
    x = hbm(x)
    Win0, Wout0 = hbm(Win0), hbm(Wout0)
    Win1, Wout1 = hbm(Win1), hbm(Wout1)
    Win2, Wout2 = hbm(Win2), hbm(Wout2)

    return pl.pallas_call(
        body,
        out_shape=jax.ShapeDtypeStruct((m, d), jnp.bfloat16),
        in_specs=[pl.BlockSpec(memory_space=pltpu.MemorySpace.HBM)] * 7,
        out_specs=pl.BlockSpec(memory_space=pltpu.MemorySpace.HBM),
        scratch_shapes=[
            pltpu.VMEM((m, d), jnp.float32),
            pltpu.VMEM((N_LAYERS, d, f), jnp.float32),
            pltpu.VMEM((N_LAYERS, f, d), jnp.float32),
            pltpu.VMEM((N_LAYERS, N_DEV, CH, m, dch), jnp.bfloat16),
            pltpu.VMEM((m, d), jnp.bfloat16),
            pltpu.SemaphoreType.DMA((2 * N_LAYERS,)),
            pltpu.SemaphoreType.DMA((2,)),
            pltpu.SemaphoreType.DMA((CH,)),
            pltpu.SemaphoreType.DMA((N_LAYERS, N_DEV - 1, CH)),
            pltpu.SemaphoreType.DMA((N_LAYERS, N_DEV, CH)),
        ],
        compiler_params=(
            pltpu.CompilerParams()
            if _ABLATE == "nocomm"
            else pltpu.CompilerParams(collective_id=0)
        ),
    )(x, Win0, Wout0, Win1, Wout1, Win2, Wout2)
